# baseline (speedup 1.0000x reference)
import functools

import jax
import jax.numpy as jnp
from jax import lax
from jax.experimental import pallas as pl
from jax.experimental.pallas import tpu as pltpu
from jax.experimental.pallas import tpu_sc as plsc

_B = 16384
_D = 128
_R = 8192

_NC = 2
_NS = 16
_NW = _NC * _NS
_RPW = _B // _NW
_CH = 128
_NCH = _RPW // _CH
_NL = 16


def _sc_body(sid_hbm, kid_hbm, cid_hbm, t1_hbm, t2_hbm, t3_hbm, out_hbm,
             sid_v, kid_v, cid_v, t1_v, t2_v, t3_v, t12_v, comb_v, fl_v,
             ob_v, ob2_v, sem):
    wid = lax.axis_index("s") * _NC + lax.axis_index("c")
    row0 = wid * _NCH

    pltpu.sync_copy(t1_hbm, t1_v)
    pltpu.sync_copy(t2_hbm, t2_v)
    pltpu.sync_copy(t3_hbm, t3_v)
    pltpu.sync_copy(sid_hbm.at[pl.ds(row0, _NCH)], sid_v)
    pltpu.sync_copy(kid_hbm.at[pl.ds(row0, _NCH)], kid_v)
    pltpu.sync_copy(cid_hbm.at[pl.ds(row0, _NCH)], cid_v)

    def t12_row(sk, _):
        s = sk // 24
        k = sk - s * 24
        for j in range(_D // _NL):
            sl = pl.ds(sk * _D + j * _NL, _NL)
            t12_v[0, sl] = (
                t1_v[0, pl.ds(s * _D + j * _NL, _NL)]
                + t2_v[0, pl.ds(k * _D + j * _NL, _NL)]
            )
        return 0

    lax.fori_loop(0, 72, t12_row, 0)

    def c_row(sk, _):
        for c in range(5):
            for j in range(_D // _NL):
                comb_v[0, pl.ds((sk * 5 + c) * _D + j * _NL, _NL)] = (
                    t12_v[0, pl.ds(sk * _D + j * _NL, _NL)]
                    + t3_v[0, pl.ds(c * _D + j * _NL, _NL)]
                )
        return 0

    lax.fori_loop(0, 72, c_row, 0)

    @plsc.parallel_loop(0, _NCH * _CH // _NL, 1)
    def flat(g):
        a = g // (_CH // _NL)
        b = g - a * (_CH // _NL)
        gsl = pl.ds(b * _NL, _NL)
        fl_v[a, gsl] = (
            (sid_v[a, gsl] * 24 + kid_v[a, gsl]) * 5 + cid_v[a, gsl]
        ) * _D

    cps = [None, None]
    for ch in range(_NCH):
        ob = (ob_v, ob2_v)[ch % 2]
        if cps[ch % 2] is not None:
            cps[ch % 2].wait()

        @plsc.parallel_loop(0, _CH // _NL, 1)
        def grp(g):
            fvec = fl_v[ch, pl.ds(g * _NL, _NL)]
            for l in range(_NL):
                off = fvec[l]
                row = g * _NL + l
                for j in range(_D // _NL):
                    ob[row, pl.ds(j * _NL, _NL)] = (
                        comb_v[0, pl.ds(off + j * _NL, _NL)]
                    )

        cps[ch % 2] = pltpu.async_copy(
            ob, out_hbm.at[pl.ds((row0 + ch) * _CH, _CH)], sem)
    for cp in cps:
        if cp is not None:
            cp.wait()


@functools.partial(
    pl.kernel,
    out_type=jax.ShapeDtypeStruct((_B, _D), jnp.float32),
    mesh=plsc.VectorSubcoreMesh(core_axis_name="c", subcore_axis_name="s"),
    scratch_types=[
        pltpu.VMEM((_NCH, _CH), jnp.int32),
        pltpu.VMEM((_NCH, _CH), jnp.int32),
        pltpu.VMEM((_NCH, _CH), jnp.int32),
        pltpu.VMEM((1, 3 * _D), jnp.float32),
        pltpu.VMEM((1, 24 * _D), jnp.float32),
        pltpu.VMEM((1, 5 * _D), jnp.float32),
        pltpu.VMEM((1, 72 * _D), jnp.float32),
        pltpu.VMEM((1, 360 * _D), jnp.float32),
        pltpu.VMEM((_NCH, _CH), jnp.int32),
        pltpu.VMEM((_CH, _D), jnp.float32),
        pltpu.VMEM((_CH, _D), jnp.float32),
        pltpu.SemaphoreType.DMA,
    ],
)
def _sc_conditioning(sid_hbm, kid_hbm, cid_hbm, t1_hbm, t2_hbm, t3_hbm,
                     out_hbm, *scratch):
    _sc_body(sid_hbm, kid_hbm, cid_hbm, t1_hbm, t2_hbm, t3_hbm, out_hbm,
             *scratch)


def _tc_body(g_ref, w_ref, b_ref, c_ref, o_ref):
    o_ref[...] = (
        jnp.dot(g_ref[...], w_ref[...], preferred_element_type=jnp.float32)
        + b_ref[...]
        + c_ref[...]
    )


def kernel(style_ids, key_ids, section_ids, groove_features, style_table,
           key_table, section_table, groove_W, groove_b):
    sid = style_ids.astype(jnp.int32).reshape(_B // _CH, _CH)
    kid = key_ids.astype(jnp.int32).reshape(_B // _CH, _CH)
    cid = section_ids.astype(jnp.int32).reshape(_B // _CH, _CH)
    cond = _sc_conditioning(
        sid, kid, cid,
        style_table.reshape(1, 3 * _D),
        key_table.reshape(1, 24 * _D),
        section_table.reshape(1, 5 * _D),
    )
    return pl.pallas_call(
        _tc_body,
        grid=(_B // _R,),
        in_specs=[
            pl.BlockSpec((_R, 32), lambda i: (i, 0)),
            pl.BlockSpec((32, _D), lambda i: (0, 0)),
            pl.BlockSpec((1, _D), lambda i: (0, 0)),
            pl.BlockSpec((_R, _D), lambda i: (i, 0)),
        ],
        out_specs=pl.BlockSpec((_R, _D), lambda i: (i, 0)),
        out_shape=jax.ShapeDtypeStruct((_B, _D), jnp.float32),
    )(groove_features, groove_W, groove_b.reshape(1, _D), cond)


# speedup vs baseline: 1.1823x; 1.1823x over previous
import functools

import jax
import jax.numpy as jnp
from jax import lax
from jax.experimental import pallas as pl
from jax.experimental.pallas import tpu as pltpu
from jax.experimental.pallas import tpu_sc as plsc

_B = 16384
_D = 128
_R = 8192

_NC = 2
_NS = 16
_NW = _NC * _NS
_RPW = _B // _NW
_CH = 128
_NCH = _RPW // _CH
_NL = 16


def _sc_body(sid_hbm, kid_hbm, cid_hbm, t1_hbm, t2_hbm, t3_hbm, out_hbm,
             sid_v, kid_v, cid_v, t1_v, t2_v, t3_v, ob_v, sem):
    wid = lax.axis_index("s") * _NC + lax.axis_index("c")
    row0 = wid * _NCH

    pltpu.sync_copy(t1_hbm, t1_v)
    pltpu.sync_copy(t2_hbm, t2_v)
    pltpu.sync_copy(t3_hbm, t3_v)
    pltpu.sync_copy(sid_hbm.at[pl.ds(row0, _NCH)], sid_v)
    pltpu.sync_copy(kid_hbm.at[pl.ds(row0, _NCH)], kid_v)
    pltpu.sync_copy(cid_hbm.at[pl.ds(row0, _NCH)], cid_v)

    def chunk(ch, _):
        @plsc.parallel_loop(0, _CH // _NL, 1)
        def grp(g):
            gsl = pl.ds(g * _NL, _NL)
            svec = sid_v[ch, gsl] * _D
            kvec = kid_v[ch, gsl] * _D
            cvec = cid_v[ch, gsl] * _D
            for l in range(_NL):
                soff = svec[l]
                koff = kvec[l]
                coff = cvec[l]
                row = g * _NL + l
                for j in range(_D // _NL):
                    sl = pl.ds(j * _NL, _NL)
                    ob_v[row, sl] = (
                        t1_v[0, pl.ds(soff + j * _NL, _NL)]
                        + t2_v[0, pl.ds(koff + j * _NL, _NL)]
                        + t3_v[0, pl.ds(coff + j * _NL, _NL)]
                    )

        pltpu.sync_copy(ob_v, out_hbm.at[pl.ds((row0 + ch) * _CH, _CH)])
        return 0

    lax.fori_loop(0, _NCH, chunk, 0)


@functools.partial(
    pl.kernel,
    out_type=jax.ShapeDtypeStruct((_B, _D), jnp.float32),
    mesh=plsc.VectorSubcoreMesh(core_axis_name="c", subcore_axis_name="s"),
    scratch_types=[
        pltpu.VMEM((_NCH, _CH), jnp.int32),
        pltpu.VMEM((_NCH, _CH), jnp.int32),
        pltpu.VMEM((_NCH, _CH), jnp.int32),
        pltpu.VMEM((1, 3 * _D), jnp.float32),
        pltpu.VMEM((1, 24 * _D), jnp.float32),
        pltpu.VMEM((1, 5 * _D), jnp.float32),
        pltpu.VMEM((_CH, _D), jnp.float32),
        pltpu.SemaphoreType.DMA,
    ],
)
def _sc_conditioning(sid_hbm, kid_hbm, cid_hbm, t1_hbm, t2_hbm, t3_hbm,
                     out_hbm, *scratch):
    _sc_body(sid_hbm, kid_hbm, cid_hbm, t1_hbm, t2_hbm, t3_hbm, out_hbm,
             *scratch)


def _tc_body(g_ref, w_ref, b_ref, c_ref, o_ref):
    o_ref[...] = (
        jnp.dot(g_ref[...], w_ref[...], preferred_element_type=jnp.float32)
        + b_ref[...]
        + c_ref[...]
    )


def kernel(style_ids, key_ids, section_ids, groove_features, style_table,
           key_table, section_table, groove_W, groove_b):
    sid = style_ids.astype(jnp.int32).reshape(_B // _CH, _CH)
    kid = key_ids.astype(jnp.int32).reshape(_B // _CH, _CH)
    cid = section_ids.astype(jnp.int32).reshape(_B // _CH, _CH)
    cond = _sc_conditioning(
        sid, kid, cid,
        style_table.reshape(1, 3 * _D),
        key_table.reshape(1, 24 * _D),
        section_table.reshape(1, 5 * _D),
    )
    return pl.pallas_call(
        _tc_body,
        grid=(_B // _R,),
        in_specs=[
            pl.BlockSpec((_R, 32), lambda i: (i, 0)),
            pl.BlockSpec((32, _D), lambda i: (0, 0)),
            pl.BlockSpec((1, _D), lambda i: (0, 0)),
            pl.BlockSpec((_R, _D), lambda i: (i, 0)),
        ],
        out_specs=pl.BlockSpec((_R, _D), lambda i: (i, 0)),
        out_shape=jax.ShapeDtypeStruct((_B, _D), jnp.float32),
    )(groove_features, groove_W, groove_b.reshape(1, _D), cond)
